# baseline (speedup 1.0000x reference)
import functools

import jax
import jax.numpy as jnp
from jax import lax
from jax.experimental import pallas as pl
from jax.experimental.pallas import tpu as pltpu
from jax.experimental.pallas import tpu_sc as plsc

_VOCAB = 100000
_DIM = 64
_B = 4096
_L_TXT = 200
_L_TOP = 20

_NC = 2
_NS = 16
_NW = _NC * _NS
_BW = _B // _NW

_RB_TXT = 2
_RB_TOP = 8
_TXT_CHUNKS = _BW // _RB_TXT
_TOP_CHUNKS = _BW // _RB_TOP
_TXT_SPLITS = ((0, 104), (104, 96))


def _pipeline(n, fire_idx, fire_gathers, fire_write, wait_write):
  def iter_block(g, p):
    q = 1 - p
    wait_write(g - 1, q)
    fire_gathers(q)
    fire_write(g, p)
    @pl.when(g + 2 <= n - 1)
    def _():
      fire_idx(g + 2, p)

  fire_idx(0, 0)
  fire_idx(1, 1)
  fire_gathers(0)
  fire_gathers(1)
  fire_write(0, 0)
  if n >= 3:
    fire_idx(2, 0)

  n_iter = n - 2
  n_pairs = n_iter // 2
  if n_pairs > 0:
    def pair_body(i, carry):
      g0 = 1 + 2 * i
      for b in range(2):
        iter_block(g0 + b, (1 + b) % 2)
      return carry

    lax.fori_loop(0, n_pairs, pair_body, 0)

  for g in range(1 + 2 * n_pairs, n - 1):
    iter_block(g, g % 2)

  p_last = (n - 1) % 2
  wait_write(n - 2, 1 - p_last)
  fire_write(n - 1, p_last)
  wait_write(n - 1, p_last)


def _run_stripe(idx_hbm, out_hbm, table_hbm, idx_v, rows_v, sems,
                b0, n_chunks, rb, gathers):
  def fire_idx(g, p):
    pltpu.async_copy(idx_hbm.at[pl.ds(b0 + g * rb, rb)], idx_v.at[p],
                     sems[p][0])

  def fire_gathers(p):
    pltpu.make_async_copy(idx_hbm.at[pl.ds(b0, rb)], idx_v.at[p],
                          sems[p][0]).wait()
    for idx_sl, dst_sl, _ in gathers(p):
      pltpu.async_copy(table_hbm.at[idx_sl], dst_sl, sems[p][1])

  def fire_write(g, p):
    for _, dst_sl, n_rows in gathers(p):
      pltpu.make_async_copy(table_hbm.at[pl.ds(0, n_rows)], dst_sl,
                            sems[p][1]).wait()
    pltpu.async_copy(rows_v.at[p], out_hbm.at[pl.ds(b0 + g * rb, rb)],
                     sems[p][2])

  def wait_write(g, p):
    pltpu.make_async_copy(rows_v.at[p], out_hbm.at[pl.ds(b0 + g * rb, rb)],
                          sems[p][2]).wait()

  _pipeline(n_chunks, fire_idx, fire_gathers, fire_write, wait_write)


_mesh = plsc.VectorSubcoreMesh(core_axis_name="c", subcore_axis_name="s")


@functools.partial(
    pl.kernel,
    mesh=_mesh,
    compiler_params=pltpu.CompilerParams(use_tc_tiling_on_sc=False),
    out_type=(
        jax.ShapeDtypeStruct((_B, _L_TXT, _DIM), jnp.float32),
        jax.ShapeDtypeStruct((_B, _L_TOP, _DIM), jnp.float32),
    ),
    scratch_types=[
        pltpu.VMEM((2, _RB_TXT, _L_TXT), jnp.int32),
        pltpu.VMEM((2, _RB_TXT, _L_TXT, _DIM), jnp.float32),
        pltpu.VMEM((2, _RB_TOP, _L_TOP), jnp.int32),
        pltpu.VMEM((2, _RB_TOP, _L_TOP, _DIM), jnp.float32),
        pltpu.SemaphoreType.DMA,
        pltpu.SemaphoreType.DMA,
        pltpu.SemaphoreType.DMA,
        pltpu.SemaphoreType.DMA,
        pltpu.SemaphoreType.DMA,
        pltpu.SemaphoreType.DMA,
    ],
)
def _embed_lookup(text_hbm, topic_hbm, table_hbm, txt_out, top_out,
                  idx_t, rows_t, idx_p, rows_p, si0, sg0, sw0, si1, sg1, sw1):
  wid = lax.axis_index("s") * _NC + lax.axis_index("c")
  b0 = wid * _BW
  sems = ((si0, sg0, sw0), (si1, sg1, sw1))

  def txt_gathers(p):
    out = []
    for r in range(_RB_TXT):
      for h, w in _TXT_SPLITS:
        out.append((idx_t.at[p, r, pl.ds(h, w)],
                    rows_t.at[p, r, pl.ds(h, w)], w))
    return out

  def top_gathers(p):
    return [(idx_p.at[p, r], rows_p.at[p, r], _L_TOP)
            for r in range(_RB_TOP)]

  _run_stripe(text_hbm, txt_out, table_hbm, idx_t, rows_t, sems,
              b0, _TXT_CHUNKS, _RB_TXT, txt_gathers)
  _run_stripe(topic_hbm, top_out, table_hbm, idx_p, rows_p, sems,
              b0, _TOP_CHUNKS, _RB_TOP, top_gathers)


def kernel(text, topic, table):
  return _embed_lookup(text.astype(jnp.int32), topic.astype(jnp.int32), table)
